# baseline (speedup 1.0000x reference)
import functools

import jax
import jax.numpy as jnp
from jax import lax
from jax.experimental import pallas as pl
from jax.experimental.pallas import tpu as pltpu
from jax.experimental.pallas import tpu_sc as plsc

LD = 112
LK = 16
ROW = 128
_CHUNK = 128
_NW = 32


def _sc_mesh():
    return plsc.VectorSubcoreMesh(core_axis_name="c", subcore_axis_name="s")


_K = 4


def _make_gather(e):
    nch = e // _CHUNK
    nj = (nch + _NW - 1) // _NW
    nj_outer = (nj + _K - 1) // _K

    @functools.partial(
        pl.kernel,
        out_type=jax.ShapeDtypeStruct((e, ROW), jnp.float32),
        mesh=_sc_mesh(),
        scratch_types=[
            [pltpu.VMEM((1, _CHUNK), jnp.int32) for _ in range(_K)],
            [pltpu.VMEM((_CHUNK, ROW), jnp.float32) for _ in range(_K)],
            [pltpu.SemaphoreType.DMA for _ in range(_K)],
            [pltpu.SemaphoreType.DMA for _ in range(_K)],
            [pltpu.SemaphoreType.DMA for _ in range(_K)],
        ],
    )
    def gk(idx_hbm, tab_hbm, out_hbm, idx_v, rows_v, si, sg, sw):
        wid = lax.axis_index("s") * 2 + lax.axis_index("c")

        def body(j, carry):
            chs = [wid + _NW * (j * _K + kk) for kk in range(_K)]
            for kk in range(_K):
                @pl.when(chs[kk] < nch)
                def _(kk=kk):
                    pltpu.async_copy(idx_hbm.at[pl.ds(chs[kk], 1)],
                                     idx_v[kk], si[kk])
            for kk in range(_K):
                @pl.when(chs[kk] < nch)
                def _(kk=kk):
                    pltpu.make_async_copy(idx_hbm.at[pl.ds(chs[kk], 1)],
                                          idx_v[kk], si[kk]).wait()
                    pltpu.async_copy(tab_hbm.at[idx_v[kk].at[0]],
                                     rows_v[kk], sg[kk])
            for kk in range(_K):
                @pl.when(chs[kk] < nch)
                def _(kk=kk):
                    pltpu.make_async_copy(tab_hbm.at[idx_v[kk].at[0]],
                                          rows_v[kk], sg[kk]).wait()
                    pltpu.async_copy(
                        rows_v[kk],
                        out_hbm.at[pl.ds(chs[kk] * _CHUNK, _CHUNK)], sw[kk])
            for kk in range(_K):
                @pl.when(chs[kk] < nch)
                def _(kk=kk):
                    pltpu.make_async_copy(
                        rows_v[kk],
                        out_hbm.at[pl.ds(chs[kk] * _CHUNK, _CHUNK)],
                        sw[kk]).wait()
            return carry

        lax.fori_loop(0, nj_outer, body, 0)

    return gk


def _make_scatter(n, e, p=ROW):
    nch = e // _CHUNK
    nj = (nch + _NW - 1) // _NW
    ks = 2
    rpt = (-(-n // 16) + 7) // 8 * 8
    rlast = n - 15 * rpt

    @functools.partial(
        pl.kernel,
        out_type=jax.ShapeDtypeStruct((2 * n, p), jnp.float32),
        mesh=_sc_mesh(),
        scratch_types=[
            [pltpu.VMEM((1, _CHUNK), jnp.int32) for _ in range(ks)],
            [pltpu.VMEM((_CHUNK, p), jnp.float32) for _ in range(ks)],
            pltpu.VMEM_SHARED((n, p), jnp.float32),
            [pltpu.SemaphoreType.DMA for _ in range(ks)],
            [pltpu.SemaphoreType.DMA for _ in range(ks)],
            [pltpu.SemaphoreType.DMA for _ in range(ks)],
        ],
    )
    def sk(idx_hbm, msg_hbm, zeros_hbm, out_hbm, idx_v, rows_v, acc_sh,
           si, sm, sa):
        cid = lax.axis_index("c")
        sid = lax.axis_index("s")
        wid = sid * 2 + cid

        @pl.when(sid < 15)
        def _():
            pltpu.sync_copy(zeros_hbm.at[pl.ds(sid * rpt, rpt)],
                            acc_sh.at[pl.ds(sid * rpt, rpt)])

        @pl.when(sid == 15)
        def _():
            pltpu.sync_copy(zeros_hbm.at[pl.ds(15 * rpt, rlast)],
                            acc_sh.at[pl.ds(15 * rpt, rlast)])

        plsc.subcore_barrier()

        def body(j, carry):
            chs = [wid + _NW * (j * ks + kk) for kk in range(ks)]
            for kk in range(ks):
                @pl.when(chs[kk] < nch)
                def _(kk=kk):
                    pltpu.async_copy(idx_hbm.at[pl.ds(chs[kk], 1)],
                                     idx_v[kk], si[kk])
                    pltpu.async_copy(
                        msg_hbm.at[pl.ds(chs[kk] * _CHUNK, _CHUNK)],
                        rows_v[kk], sm[kk])
            for kk in range(ks):
                @pl.when(chs[kk] < nch)
                def _(kk=kk):
                    pltpu.make_async_copy(idx_hbm.at[pl.ds(chs[kk], 1)],
                                          idx_v[kk], si[kk]).wait()
                    pltpu.make_async_copy(
                        msg_hbm.at[pl.ds(chs[kk] * _CHUNK, _CHUNK)],
                        rows_v[kk], sm[kk]).wait()
                    pltpu.async_copy(rows_v[kk], acc_sh.at[idx_v[kk].at[0]],
                                     sa[kk], add=True)
            for kk in range(ks):
                @pl.when(chs[kk] < nch)
                def _(kk=kk):
                    pltpu.make_async_copy(rows_v[kk],
                                          acc_sh.at[idx_v[kk].at[0]],
                                          sa[kk]).wait()
            return carry

        lax.fori_loop(0, (nj + ks - 1) // ks, body, 0)
        plsc.subcore_barrier()

        @pl.when(sid < 15)
        def _():
            pltpu.sync_copy(acc_sh.at[pl.ds(sid * rpt, rpt)],
                            out_hbm.at[pl.ds(cid * n + sid * rpt, rpt)])

        @pl.when(sid == 15)
        def _():
            pltpu.sync_copy(acc_sh.at[pl.ds(15 * rpt, rlast)],
                            out_hbm.at[pl.ds(cid * n + 15 * rpt, rlast)])

    return sk


def _edge_body(ys_ref, g_ref, ea_ref, tabt_ref, tab_ref, w2k_ref, w2kt_ref,
               we2o_ref, b_ref, wa2_ref, mv_ref, msg_ref, *, c, p):
    ysv = ys_ref[:, 0:LD]
    z = g_ref[:, 0:8]
    qb = g_ref[:, 8:8 + LK]
    ve = g_ref[:, 8 + LK:8 + LK + 8]
    ed = jnp.sum(ea_ref[...] * ve, axis=1, keepdims=True)
    u = jnp.dot(z, tabt_ref[...], preferred_element_type=jnp.float32)
    xl = (jnp.dot(ysv * u, w2k_ref[...], preferred_element_type=jnp.float32)
          + qb + ed) * 0.25
    xl = xl - jnp.max(xl, axis=1, keepdims=True)
    exl = jnp.exp(xl)
    alpha = exl / jnp.sum(exl, axis=1, keepdims=True)
    r = jnp.dot(alpha, w2kt_ref[...], preferred_element_type=jnp.float32)
    h8 = jnp.dot(ysv * r, tab_ref[...], preferred_element_type=jnp.float32)
    out = (jnp.dot(h8, we2o_ref[...], preferred_element_type=jnp.float32)
           + jnp.dot(alpha, b_ref[...], preferred_element_type=jnp.float32))
    s = jnp.dot(out, wa2_ref[...], preferred_element_type=jnp.float32)
    w = jnp.exp(s - mv_ref[0, 0])
    pad = jnp.zeros((out.shape[0], p - c - 1), jnp.float32)
    msg_ref[...] = jnp.concatenate([w, w * out, pad], axis=1)


def _edge_call(ys, g, ea, f, p, eb=4000):
    e = ys.shape[0]
    c = f["c"]
    return pl.pallas_call(
        functools.partial(_edge_body, c=c, p=p),
        grid=(e // eb,),
        in_specs=[
            pl.BlockSpec((eb, ROW), lambda i: (i, 0)),
            pl.BlockSpec((eb, ROW), lambda i: (i, 0)),
            pl.BlockSpec((eb, 8), lambda i: (i, 0)),
            pl.BlockSpec((8, LD), lambda i: (0, 0)),
            pl.BlockSpec((LD, 8), lambda i: (0, 0)),
            pl.BlockSpec((LD, LK), lambda i: (0, 0)),
            pl.BlockSpec((LK, LD), lambda i: (0, 0)),
            pl.BlockSpec((8, c), lambda i: (0, 0)),
            pl.BlockSpec((LK, c), lambda i: (0, 0)),
            pl.BlockSpec((c, 1), lambda i: (0, 0)),
            pl.BlockSpec((1, 1), lambda i: (0, 0)),
        ],
        out_specs=pl.BlockSpec((eb, p), lambda i: (i, 0)),
        out_shape=jax.ShapeDtypeStruct((e, p), jnp.float32),
    )(ys, g, ea, f["tabt"], f["tab"], f["w2k"], f["w2kt"], f["we2o"],
      f["b"], f["wa2"], f["mv"])


def _proj_body(x_ref, w_ref, b_ref, o_ref):
    o_ref[...] = (jnp.dot(x_ref[...], w_ref[...],
                          preferred_element_type=jnp.float32) + b_ref[...])


def _proj_call(x, w, b, nb=2000):
    n, din = x.shape
    d = w.shape[1]
    return pl.pallas_call(
        _proj_body,
        grid=(n // nb,),
        in_specs=[
            pl.BlockSpec((nb, din), lambda i: (i, 0)),
            pl.BlockSpec((din, d), lambda i: (0, 0)),
            pl.BlockSpec((1, d), lambda i: (0, 0)),
        ],
        out_specs=pl.BlockSpec((nb, d), lambda i: (i, 0)),
        out_shape=jax.ShapeDtypeStruct((n, d), jnp.float32),
    )(x, w, b)


def _combine_call(x, acc, wsc, wcb, bf, wu, bu, c, p, relu, nb=2000):
    n, din = x.shape
    proj = wu is not None
    if not proj:
        wu = jnp.zeros((c, 8), jnp.float32)
        bu = jnp.zeros((1, 8), jnp.float32)
    du = wu.shape[1]
    nblk = n // nb

    def body(x_ref, a0_ref, a1_ref, wsc_ref, wcb_ref, bf_ref, wu_ref, bu_ref,
             *outs):
        den = a0_ref[:, 0:1] + a1_ref[:, 0:1]
        num = a0_ref[:, 1:c + 1] + a1_ref[:, 1:c + 1]
        m = num / (den + 1e-16)
        xn = (jnp.dot(x_ref[...], wsc_ref[...],
                      preferred_element_type=jnp.float32)
              + jnp.dot(m, wcb_ref[...], preferred_element_type=jnp.float32)
              + bf_ref[...])
        if relu:
            xn = jnp.maximum(xn, 0.0)
        outs[0][...] = xn
        if proj:
            outs[1][...] = (jnp.dot(xn, wu_ref[...],
                                    preferred_element_type=jnp.float32)
                            + bu_ref[...])

    out_shape = [jax.ShapeDtypeStruct((n, c), jnp.float32)]
    out_specs = [pl.BlockSpec((nb, c), lambda i: (i, 0))]
    if proj:
        out_shape.append(jax.ShapeDtypeStruct((n, du), jnp.float32))
        out_specs.append(pl.BlockSpec((nb, du), lambda i: (i, 0)))
    return pl.pallas_call(
        body,
        grid=(nblk,),
        in_specs=[
            pl.BlockSpec((nb, din), lambda i: (i, 0)),
            pl.BlockSpec((nb, p), lambda i: (i, 0)),
            pl.BlockSpec((nb, p), lambda i: (nblk + i, 0)),
            pl.BlockSpec((din, c), lambda i: (0, 0)),
            pl.BlockSpec((c, c), lambda i: (0, 0)),
            pl.BlockSpec((1, c), lambda i: (0, 0)),
            pl.BlockSpec((wu.shape[0], du), lambda i: (0, 0)),
            pl.BlockSpec((1, du), lambda i: (0, 0)),
        ],
        out_specs=out_specs,
        out_shape=out_shape,
    )(x, acc, acc, wsc, wcb, bf, wu, bu)


def _fold(p, table, y):
    c = p["Wq"].shape[1]
    t2 = table @ p["Wemb2out"]
    b = jnp.outer(p["bin2k"], p["Wemb2out"].sum(0)) + p["bemb2out"][None, :]
    ek = p["Wemb2out"] @ p["Wkkey"]
    b2 = b @ p["Wkkey"] + p["bkkey"][None, :]
    wa2 = p["Walpha"][c:, 0]
    t2w = t2 @ wa2
    bw = b @ wa2
    klwa = y @ (p["Win2k"] * t2w[:, None]) + bw[None, :]
    mshift = jnp.max(klwa).reshape(1, 1)
    b2p = b2 + p["bedge"][None, :]
    wbig = jnp.concatenate([ek.T, b2p.T, p["Wedge"].T,
                            jnp.zeros((c, ROW - 32), jnp.float32)],
                           axis=1)
    wu = p["Wq"] @ wbig
    bu = (p["bq"] @ wbig).reshape(1, ROW)
    wsc = p["Wskip"] @ p["Wcomb"][:c]
    bf = (p["bskip"] @ p["Wcomb"][:c] + p["bcomb"]).reshape(1, c)
    wcb = p["Wcomb"][c:]
    return dict(c=c, w2k=p["Win2k"], w2kt=p["Win2k"].T, tab=table,
                tabt=table.T, we2o=p["Wemb2out"], b=b,
                wa2=wa2.reshape(c, 1), mv=mshift, wu=wu, bu=bu,
                wsc=wsc, bf=bf, wcb=wcb)


def kernel(features, edge_index, edge_attr, y, eval_mask, table, layers):
    n = features.shape[0]
    e = edge_index.shape[1]
    nch = e // _CHUNK
    src2d = edge_index[0].reshape(nch, _CHUNK)
    dst2d = edge_index[1].reshape(nch, _CHUNK)
    y128 = jnp.pad(y, ((0, 0), (0, ROW - y.shape[1])))

    folds = [_fold(p, table, y) for p in layers]
    gather = _make_gather(e)
    ys = gather(src2d, y128)

    x = features
    for li, f in enumerate(folds):
        c = f["c"]
        p = ROW
        last = li == len(folds) - 1
        if li == 0:
            u_nodes = _proj_call(features, f["wu"], f["bu"])
        g = gather(dst2d, u_nodes)
        msg = _edge_call(ys, g, edge_attr, f, p)
        acc = _make_scatter(n, e, p)(dst2d, msg, jnp.zeros((n, p), jnp.float32))
        nxt = None if last else folds[li + 1]
        res = _combine_call(x, acc, f["wsc"], f["wcb"], f["bf"],
                            None if last else nxt["wu"],
                            None if last else nxt["bu"],
                            c, p, relu=not last)
        if last:
            x = res[0]
        else:
            x, u_nodes = res
    return x


# speedup vs baseline: 1.1038x; 1.1038x over previous
import functools

import jax
import jax.numpy as jnp
from jax import lax
from jax.experimental import pallas as pl
from jax.experimental.pallas import tpu as pltpu
from jax.experimental.pallas import tpu_sc as plsc

LD = 112
LK = 16
ROW = 128
_CHUNK = 128
_NW = 32


def _sc_mesh():
    return plsc.VectorSubcoreMesh(core_axis_name="c", subcore_axis_name="s")


_K = 4


def _make_gather(e):
    nch = e // _CHUNK
    nj = (nch + _NW - 1) // _NW
    nj_outer = (nj + _K - 1) // _K

    @functools.partial(
        pl.kernel,
        out_type=jax.ShapeDtypeStruct((e, ROW), jnp.float32),
        mesh=_sc_mesh(),
        scratch_types=[
            [pltpu.VMEM((1, _CHUNK), jnp.int32) for _ in range(_K)],
            [pltpu.VMEM((_CHUNK, ROW), jnp.float32) for _ in range(_K)],
            [pltpu.SemaphoreType.DMA for _ in range(_K)],
            [pltpu.SemaphoreType.DMA for _ in range(_K)],
            [pltpu.SemaphoreType.DMA for _ in range(_K)],
        ],
    )
    def gk(idx_hbm, tab_hbm, out_hbm, idx_v, rows_v, si, sg, sw):
        wid = lax.axis_index("s") * 2 + lax.axis_index("c")

        def body(j, carry):
            chs = [wid + _NW * (j * _K + kk) for kk in range(_K)]
            for kk in range(_K):
                @pl.when(chs[kk] < nch)
                def _(kk=kk):
                    pltpu.async_copy(idx_hbm.at[pl.ds(chs[kk], 1)],
                                     idx_v[kk], si[kk])
            for kk in range(_K):
                @pl.when(chs[kk] < nch)
                def _(kk=kk):
                    pltpu.make_async_copy(idx_hbm.at[pl.ds(chs[kk], 1)],
                                          idx_v[kk], si[kk]).wait()
                    pltpu.async_copy(tab_hbm.at[idx_v[kk].at[0]],
                                     rows_v[kk], sg[kk])
            for kk in range(_K):
                @pl.when(chs[kk] < nch)
                def _(kk=kk):
                    pltpu.make_async_copy(tab_hbm.at[idx_v[kk].at[0]],
                                          rows_v[kk], sg[kk]).wait()
                    pltpu.async_copy(
                        rows_v[kk],
                        out_hbm.at[pl.ds(chs[kk] * _CHUNK, _CHUNK)], sw[kk])
            for kk in range(_K):
                @pl.when(chs[kk] < nch)
                def _(kk=kk):
                    pltpu.make_async_copy(
                        rows_v[kk],
                        out_hbm.at[pl.ds(chs[kk] * _CHUNK, _CHUNK)],
                        sw[kk]).wait()
            return carry

        lax.fori_loop(0, nj_outer, body, 0)

    return gk


def _make_scatter(n, e, p=ROW):
    nch = e // _CHUNK
    nj = (nch + _NW - 1) // _NW
    ks = 2
    rpt = (-(-n // 16) + 7) // 8 * 8
    rlast = n - 15 * rpt

    @functools.partial(
        pl.kernel,
        out_type=jax.ShapeDtypeStruct((2 * n, p), jnp.float32),
        mesh=_sc_mesh(),
        scratch_types=[
            [pltpu.VMEM((1, _CHUNK), jnp.int32) for _ in range(ks)],
            [pltpu.VMEM((_CHUNK, p), jnp.float32) for _ in range(ks)],
            pltpu.VMEM_SHARED((n, p), jnp.float32),
            [pltpu.SemaphoreType.DMA for _ in range(ks)],
            [pltpu.SemaphoreType.DMA for _ in range(ks)],
            [pltpu.SemaphoreType.DMA for _ in range(ks)],
        ],
    )
    def sk(idx_hbm, msg_hbm, zeros_hbm, out_hbm, idx_v, rows_v, acc_sh,
           si, sm, sa):
        cid = lax.axis_index("c")
        sid = lax.axis_index("s")
        wid = sid * 2 + cid

        @pl.when(sid < 15)
        def _():
            pltpu.sync_copy(zeros_hbm.at[pl.ds(sid * rpt, rpt)],
                            acc_sh.at[pl.ds(sid * rpt, rpt)])

        @pl.when(sid == 15)
        def _():
            pltpu.sync_copy(zeros_hbm.at[pl.ds(15 * rpt, rlast)],
                            acc_sh.at[pl.ds(15 * rpt, rlast)])

        plsc.subcore_barrier()

        def body(j, carry):
            chs = [wid + _NW * (j * ks + kk) for kk in range(ks)]
            for kk in range(ks):
                @pl.when(chs[kk] < nch)
                def _(kk=kk):
                    pltpu.async_copy(idx_hbm.at[pl.ds(chs[kk], 1)],
                                     idx_v[kk], si[kk])
                    pltpu.async_copy(
                        msg_hbm.at[pl.ds(chs[kk] * _CHUNK, _CHUNK)],
                        rows_v[kk], sm[kk])
            for kk in range(ks):
                @pl.when(chs[kk] < nch)
                def _(kk=kk):
                    pltpu.make_async_copy(idx_hbm.at[pl.ds(chs[kk], 1)],
                                          idx_v[kk], si[kk]).wait()
                    pltpu.make_async_copy(
                        msg_hbm.at[pl.ds(chs[kk] * _CHUNK, _CHUNK)],
                        rows_v[kk], sm[kk]).wait()
                    pltpu.async_copy(rows_v[kk], acc_sh.at[idx_v[kk].at[0]],
                                     sa[kk], add=True)
            for kk in range(ks):
                @pl.when(chs[kk] < nch)
                def _(kk=kk):
                    pltpu.make_async_copy(rows_v[kk],
                                          acc_sh.at[idx_v[kk].at[0]],
                                          sa[kk]).wait()
            return carry

        lax.fori_loop(0, (nj + ks - 1) // ks, body, 0)
        plsc.subcore_barrier()

        @pl.when(sid < 15)
        def _():
            pltpu.sync_copy(acc_sh.at[pl.ds(sid * rpt, rpt)],
                            out_hbm.at[pl.ds(cid * n + sid * rpt, rpt)])

        @pl.when(sid == 15)
        def _():
            pltpu.sync_copy(acc_sh.at[pl.ds(15 * rpt, rlast)],
                            out_hbm.at[pl.ds(cid * n + 15 * rpt, rlast)])

    return sk


def _edge_body(ys_ref, g_ref, ea_ref, tabt_ref, tab_ref, w2ke_ref,
               wcat_ref, wo_ref, msg_ref, *, c, p):
    ysv = ys_ref[:, 0:LD]
    g = g_ref[...]
    z = g[:, 0:8]
    qb = g[:, 8:8 + LK]
    ve = g[:, 8 + LK:8 + LK + 8]
    u = jnp.dot(z, tabt_ref[...], preferred_element_type=jnp.float32)
    x2 = jnp.concatenate([ysv * u, ea_ref[...] * ve], axis=1)
    xl = jnp.dot(x2, w2ke_ref[...], preferred_element_type=jnp.float32) + qb
    exl = jnp.exp(xl)
    alpha = exl / jnp.sum(exl, axis=1, keepdims=True)
    rab = jnp.dot(alpha, wcat_ref[...], preferred_element_type=jnp.float32)
    r = rab[:, 0:LD]
    h8 = jnp.dot(ysv * r, tab_ref[...], preferred_element_type=jnp.float32)
    os = (jnp.dot(h8, wo_ref[...], preferred_element_type=jnp.float32)
          + rab[:, LD:LD + 1 + c])
    w = jnp.exp(os[:, 0:1])
    lane = jax.lax.broadcasted_iota(jnp.int32, (os.shape[0], 1 + c), 1)
    base = jnp.where(lane == 0, 1.0, os)
    msg_ref[:, 0:1 + c] = w * base
    msg_ref[:, 1 + c:p] = jnp.zeros((os.shape[0], p - 1 - c), jnp.float32)


def _edge_call(ys, g, ea, f, p, eb=4000):
    e = ys.shape[0]
    c = f["c"]
    return pl.pallas_call(
        functools.partial(_edge_body, c=c, p=p),
        grid=(e // eb,),
        in_specs=[
            pl.BlockSpec((eb, ROW), lambda i: (i, 0)),
            pl.BlockSpec((eb, ROW), lambda i: (i, 0)),
            pl.BlockSpec((eb, 8), lambda i: (i, 0)),
            pl.BlockSpec((8, LD), lambda i: (0, 0)),
            pl.BlockSpec((LD, 8), lambda i: (0, 0)),
            pl.BlockSpec((LD + 8, LK), lambda i: (0, 0)),
            pl.BlockSpec((LK, LD + 1 + c), lambda i: (0, 0)),
            pl.BlockSpec((8, 1 + c), lambda i: (0, 0)),
        ],
        out_specs=pl.BlockSpec((eb, p), lambda i: (i, 0)),
        out_shape=jax.ShapeDtypeStruct((e, p), jnp.float32),
    )(ys, g, ea, f["tabt"], f["tab"], f["w2ke"], f["wcat"], f["wo"])


def _proj_body(x_ref, w_ref, b_ref, o_ref):
    o_ref[...] = (jnp.dot(x_ref[...], w_ref[...],
                          preferred_element_type=jnp.float32) + b_ref[...])


def _proj_call(x, w, b, nb=2000):
    n, din = x.shape
    d = w.shape[1]
    return pl.pallas_call(
        _proj_body,
        grid=(n // nb,),
        in_specs=[
            pl.BlockSpec((nb, din), lambda i: (i, 0)),
            pl.BlockSpec((din, d), lambda i: (0, 0)),
            pl.BlockSpec((1, d), lambda i: (0, 0)),
        ],
        out_specs=pl.BlockSpec((nb, d), lambda i: (i, 0)),
        out_shape=jax.ShapeDtypeStruct((n, d), jnp.float32),
    )(x, w, b)


def _combine_call(x, acc, wsc, wcb, bf, wu, bu, c, p, relu, nb=2000):
    n, din = x.shape
    proj = wu is not None
    if not proj:
        wu = jnp.zeros((c, 8), jnp.float32)
        bu = jnp.zeros((1, 8), jnp.float32)
    du = wu.shape[1]
    nblk = n // nb

    def body(x_ref, a0_ref, a1_ref, wsc_ref, wcb_ref, bf_ref, wu_ref, bu_ref,
             *outs):
        den = a0_ref[:, 0:1] + a1_ref[:, 0:1]
        num = a0_ref[:, 1:c + 1] + a1_ref[:, 1:c + 1]
        m = num / (den + 1e-16)
        xn = (jnp.dot(x_ref[...], wsc_ref[...],
                      preferred_element_type=jnp.float32)
              + jnp.dot(m, wcb_ref[...], preferred_element_type=jnp.float32)
              + bf_ref[...])
        if relu:
            xn = jnp.maximum(xn, 0.0)
        outs[0][...] = xn
        if proj:
            outs[1][...] = (jnp.dot(xn, wu_ref[...],
                                    preferred_element_type=jnp.float32)
                            + bu_ref[...])

    out_shape = [jax.ShapeDtypeStruct((n, c), jnp.float32)]
    out_specs = [pl.BlockSpec((nb, c), lambda i: (i, 0))]
    if proj:
        out_shape.append(jax.ShapeDtypeStruct((n, du), jnp.float32))
        out_specs.append(pl.BlockSpec((nb, du), lambda i: (i, 0)))
    return pl.pallas_call(
        body,
        grid=(nblk,),
        in_specs=[
            pl.BlockSpec((nb, din), lambda i: (i, 0)),
            pl.BlockSpec((nb, p), lambda i: (i, 0)),
            pl.BlockSpec((nb, p), lambda i: (nblk + i, 0)),
            pl.BlockSpec((din, c), lambda i: (0, 0)),
            pl.BlockSpec((c, c), lambda i: (0, 0)),
            pl.BlockSpec((1, c), lambda i: (0, 0)),
            pl.BlockSpec((wu.shape[0], du), lambda i: (0, 0)),
            pl.BlockSpec((1, du), lambda i: (0, 0)),
        ],
        out_specs=out_specs,
        out_shape=out_shape,
    )(x, acc, acc, wsc, wcb, bf, wu, bu)


def _fold(p, table, y):
    c = p["Wq"].shape[1]
    t2 = table @ p["Wemb2out"]
    b = jnp.outer(p["bin2k"], p["Wemb2out"].sum(0)) + p["bemb2out"][None, :]
    ek = p["Wemb2out"] @ p["Wkkey"]
    b2 = b @ p["Wkkey"] + p["bkkey"][None, :]
    wa2 = p["Walpha"][c:, 0]
    t2w = t2 @ wa2
    bw = b @ wa2
    klwa = y @ (p["Win2k"] * t2w[:, None]) + bw[None, :]
    mshift = jnp.max(klwa)
    b2p = (b2 + p["bedge"][None, :]) * 0.25
    wbig = jnp.concatenate([ek.T, b2p.T, p["Wedge"].T * 0.25,
                            jnp.zeros((c, ROW - 32), jnp.float32)],
                           axis=1)
    wu = p["Wq"] @ wbig
    bu = (p["bq"] @ wbig).reshape(1, ROW)
    w2ke = jnp.concatenate([p["Win2k"] * 0.25,
                            jnp.ones((8, LK), jnp.float32)], axis=0)
    wo = jnp.concatenate([(p["Wemb2out"] @ wa2)[:, None],
                          p["Wemb2out"]], axis=1)
    bo = jnp.concatenate([(b @ wa2)[:, None] - mshift, b], axis=1)
    wcat = jnp.concatenate([p["Win2k"].T, bo], axis=1)
    wsc = p["Wskip"] @ p["Wcomb"][:c]
    bf = (p["bskip"] @ p["Wcomb"][:c] + p["bcomb"]).reshape(1, c)
    wcb = p["Wcomb"][c:]
    return dict(c=c, tab=table, tabt=table.T, w2ke=w2ke, wo=wo, wcat=wcat,
                wu=wu, bu=bu, wsc=wsc, bf=bf, wcb=wcb)


def kernel(features, edge_index, edge_attr, y, eval_mask, table, layers):
    n = features.shape[0]
    e = edge_index.shape[1]
    nch = e // _CHUNK
    src2d = edge_index[0].reshape(nch, _CHUNK)
    dst2d = edge_index[1].reshape(nch, _CHUNK)
    y128 = jnp.pad(y, ((0, 0), (0, ROW - y.shape[1])))

    folds = [_fold(p, table, y) for p in layers]
    gather = _make_gather(e)
    ys = gather(src2d, y128)

    x = features
    for li, f in enumerate(folds):
        c = f["c"]
        p = ROW
        last = li == len(folds) - 1
        if li == 0:
            u_nodes = _proj_call(features, f["wu"], f["bu"])
        g = gather(dst2d, u_nodes)
        msg = _edge_call(ys, g, edge_attr, f, p)
        acc = _make_scatter(n, e, p)(dst2d, msg, jnp.zeros((n, p), jnp.float32))
        nxt = None if last else folds[li + 1]
        res = _combine_call(x, acc, f["wsc"], f["wcb"], f["bf"],
                            None if last else nxt["wu"],
                            None if last else nxt["bu"],
                            c, p, relu=not last)
        if last:
            x = res[0]
        else:
            x, u_nodes = res
    return x


# speedup vs baseline: 1.1356x; 1.0288x over previous
import functools

import jax
import jax.numpy as jnp
from jax import lax
from jax.experimental import pallas as pl
from jax.experimental.pallas import tpu as pltpu
from jax.experimental.pallas import tpu_sc as plsc

LD = 112
LK = 16
ROW = 128
_CHUNK = 128
_NW = 32


def _sc_mesh():
    return plsc.VectorSubcoreMesh(core_axis_name="c", subcore_axis_name="s")


_K = 6


def _make_gather(e):
    nch = e // _CHUNK
    nj = (nch + _NW - 1) // _NW
    nj_outer = (nj + _K - 1) // _K

    @functools.partial(
        pl.kernel,
        out_type=jax.ShapeDtypeStruct((e, ROW), jnp.float32),
        mesh=_sc_mesh(),
        scratch_types=[
            [pltpu.VMEM((1, _CHUNK), jnp.int32) for _ in range(_K)],
            [pltpu.VMEM((_CHUNK, ROW), jnp.float32) for _ in range(_K)],
            [pltpu.SemaphoreType.DMA for _ in range(_K)],
            [pltpu.SemaphoreType.DMA for _ in range(_K)],
            [pltpu.SemaphoreType.DMA for _ in range(_K)],
        ],
    )
    def gk(idx_hbm, tab_hbm, out_hbm, idx_v, rows_v, si, sg, sw):
        wid = lax.axis_index("s") * 2 + lax.axis_index("c")

        def body(j, carry):
            chs = [wid + _NW * (j * _K + kk) for kk in range(_K)]
            for kk in range(_K):
                @pl.when(chs[kk] < nch)
                def _(kk=kk):
                    pltpu.async_copy(idx_hbm.at[pl.ds(chs[kk], 1)],
                                     idx_v[kk], si[kk])
            for kk in range(_K):
                @pl.when(chs[kk] < nch)
                def _(kk=kk):
                    pltpu.make_async_copy(idx_hbm.at[pl.ds(chs[kk], 1)],
                                          idx_v[kk], si[kk]).wait()
                    pltpu.async_copy(tab_hbm.at[idx_v[kk].at[0]],
                                     rows_v[kk], sg[kk])
            for kk in range(_K):
                @pl.when(chs[kk] < nch)
                def _(kk=kk):
                    pltpu.make_async_copy(tab_hbm.at[idx_v[kk].at[0]],
                                          rows_v[kk], sg[kk]).wait()
                    pltpu.async_copy(
                        rows_v[kk],
                        out_hbm.at[pl.ds(chs[kk] * _CHUNK, _CHUNK)], sw[kk])
            for kk in range(_K):
                @pl.when(chs[kk] < nch)
                def _(kk=kk):
                    pltpu.make_async_copy(
                        rows_v[kk],
                        out_hbm.at[pl.ds(chs[kk] * _CHUNK, _CHUNK)],
                        sw[kk]).wait()
            return carry

        lax.fori_loop(0, nj_outer, body, 0)

    return gk


def _make_scatter(n, e, p=ROW):
    nch = e // _CHUNK
    nj = (nch + _NW - 1) // _NW
    ks = 3
    rpt = (-(-n // 16) + 7) // 8 * 8
    rlast = n - 15 * rpt

    @functools.partial(
        pl.kernel,
        out_type=jax.ShapeDtypeStruct((2 * n, p), jnp.float32),
        mesh=_sc_mesh(),
        scratch_types=[
            [pltpu.VMEM((1, _CHUNK), jnp.int32) for _ in range(ks)],
            [pltpu.VMEM((_CHUNK, p), jnp.float32) for _ in range(ks)],
            pltpu.VMEM_SHARED((n, p), jnp.float32),
            [pltpu.SemaphoreType.DMA for _ in range(ks)],
            [pltpu.SemaphoreType.DMA for _ in range(ks)],
            [pltpu.SemaphoreType.DMA for _ in range(ks)],
        ],
    )
    def sk(idx_hbm, msg_hbm, zeros_hbm, out_hbm, idx_v, rows_v, acc_sh,
           si, sm, sa):
        cid = lax.axis_index("c")
        sid = lax.axis_index("s")
        wid = sid * 2 + cid

        @pl.when(sid < 15)
        def _():
            pltpu.sync_copy(zeros_hbm.at[pl.ds(sid * rpt, rpt)],
                            acc_sh.at[pl.ds(sid * rpt, rpt)])

        @pl.when(sid == 15)
        def _():
            pltpu.sync_copy(zeros_hbm.at[pl.ds(15 * rpt, rlast)],
                            acc_sh.at[pl.ds(15 * rpt, rlast)])

        plsc.subcore_barrier()

        def body(j, carry):
            chs = [wid + _NW * (j * ks + kk) for kk in range(ks)]
            for kk in range(ks):
                @pl.when(chs[kk] < nch)
                def _(kk=kk):
                    pltpu.async_copy(idx_hbm.at[pl.ds(chs[kk], 1)],
                                     idx_v[kk], si[kk])
                    pltpu.async_copy(
                        msg_hbm.at[pl.ds(chs[kk] * _CHUNK, _CHUNK)],
                        rows_v[kk], sm[kk])
            for kk in range(ks):
                @pl.when(chs[kk] < nch)
                def _(kk=kk):
                    pltpu.make_async_copy(idx_hbm.at[pl.ds(chs[kk], 1)],
                                          idx_v[kk], si[kk]).wait()
                    pltpu.make_async_copy(
                        msg_hbm.at[pl.ds(chs[kk] * _CHUNK, _CHUNK)],
                        rows_v[kk], sm[kk]).wait()
                    pltpu.async_copy(rows_v[kk], acc_sh.at[idx_v[kk].at[0]],
                                     sa[kk], add=True)
            for kk in range(ks):
                @pl.when(chs[kk] < nch)
                def _(kk=kk):
                    pltpu.make_async_copy(rows_v[kk],
                                          acc_sh.at[idx_v[kk].at[0]],
                                          sa[kk]).wait()
            return carry

        lax.fori_loop(0, (nj + ks - 1) // ks, body, 0)
        plsc.subcore_barrier()

        @pl.when(sid < 15)
        def _():
            pltpu.sync_copy(acc_sh.at[pl.ds(sid * rpt, rpt)],
                            out_hbm.at[pl.ds(cid * n + sid * rpt, rpt)])

        @pl.when(sid == 15)
        def _():
            pltpu.sync_copy(acc_sh.at[pl.ds(15 * rpt, rlast)],
                            out_hbm.at[pl.ds(cid * n + 15 * rpt, rlast)])

    return sk


def _edge_body(ys_ref, g_ref, ea_ref, tabt_ref, tab_ref, w2ke_ref,
               wcat_ref, wo_ref, msg_ref, *, c, p):
    ysv = ys_ref[:, 0:LD]
    g = g_ref[...]
    z = g[:, 0:8]
    qb = g[:, 8:8 + LK]
    ve = g[:, 8 + LK:8 + LK + 8]
    u = jnp.dot(z, tabt_ref[...], preferred_element_type=jnp.float32)
    x2 = jnp.concatenate([ysv * u, ea_ref[...] * ve], axis=1)
    xl = jnp.dot(x2, w2ke_ref[...], preferred_element_type=jnp.float32) + qb
    exl = jnp.exp(xl)
    alpha = exl / jnp.sum(exl, axis=1, keepdims=True)
    rab = jnp.dot(alpha, wcat_ref[...], preferred_element_type=jnp.float32)
    r = rab[:, 0:LD]
    h8 = jnp.dot(ysv * r, tab_ref[...], preferred_element_type=jnp.float32)
    os = (jnp.dot(h8, wo_ref[...], preferred_element_type=jnp.float32)
          + rab[:, LD:LD + 1 + c])
    w = jnp.exp(os[:, 0:1])
    lane = jax.lax.broadcasted_iota(jnp.int32, (os.shape[0], 1 + c), 1)
    base = jnp.where(lane == 0, 1.0, os)
    msg_ref[:, 0:1 + c] = w * base
    msg_ref[:, 1 + c:p] = jnp.zeros((os.shape[0], p - 1 - c), jnp.float32)


def _edge_call(ys, g, ea, f, p, eb=8000):
    e = ys.shape[0]
    c = f["c"]
    return pl.pallas_call(
        functools.partial(_edge_body, c=c, p=p),
        grid=(e // eb,),
        in_specs=[
            pl.BlockSpec((eb, ROW), lambda i: (i, 0)),
            pl.BlockSpec((eb, ROW), lambda i: (i, 0)),
            pl.BlockSpec((eb, 8), lambda i: (i, 0)),
            pl.BlockSpec((8, LD), lambda i: (0, 0)),
            pl.BlockSpec((LD, 8), lambda i: (0, 0)),
            pl.BlockSpec((LD + 8, LK), lambda i: (0, 0)),
            pl.BlockSpec((LK, LD + 1 + c), lambda i: (0, 0)),
            pl.BlockSpec((8, 1 + c), lambda i: (0, 0)),
        ],
        out_specs=pl.BlockSpec((eb, p), lambda i: (i, 0)),
        out_shape=jax.ShapeDtypeStruct((e, p), jnp.float32),
    )(ys, g, ea, f["tabt"], f["tab"], f["w2ke"], f["wcat"], f["wo"])


def _proj_body(x_ref, w_ref, b_ref, o_ref):
    o_ref[...] = (jnp.dot(x_ref[...], w_ref[...],
                          preferred_element_type=jnp.float32) + b_ref[...])


def _proj_call(x, w, b, nb=2000):
    n, din = x.shape
    d = w.shape[1]
    return pl.pallas_call(
        _proj_body,
        grid=(n // nb,),
        in_specs=[
            pl.BlockSpec((nb, din), lambda i: (i, 0)),
            pl.BlockSpec((din, d), lambda i: (0, 0)),
            pl.BlockSpec((1, d), lambda i: (0, 0)),
        ],
        out_specs=pl.BlockSpec((nb, d), lambda i: (i, 0)),
        out_shape=jax.ShapeDtypeStruct((n, d), jnp.float32),
    )(x, w, b)


def _combine_call(x, acc, wsc, wcb, bf, wu, bu, c, p, relu, nb=2000):
    n, din = x.shape
    proj = wu is not None
    if not proj:
        wu = jnp.zeros((c, 8), jnp.float32)
        bu = jnp.zeros((1, 8), jnp.float32)
    du = wu.shape[1]
    nblk = n // nb

    def body(x_ref, a0_ref, a1_ref, wsc_ref, wcb_ref, bf_ref, wu_ref, bu_ref,
             *outs):
        den = a0_ref[:, 0:1] + a1_ref[:, 0:1]
        num = a0_ref[:, 1:c + 1] + a1_ref[:, 1:c + 1]
        m = num / (den + 1e-16)
        xn = (jnp.dot(x_ref[...], wsc_ref[...],
                      preferred_element_type=jnp.float32)
              + jnp.dot(m, wcb_ref[...], preferred_element_type=jnp.float32)
              + bf_ref[...])
        if relu:
            xn = jnp.maximum(xn, 0.0)
        outs[0][...] = xn
        if proj:
            outs[1][...] = (jnp.dot(xn, wu_ref[...],
                                    preferred_element_type=jnp.float32)
                            + bu_ref[...])

    out_shape = [jax.ShapeDtypeStruct((n, c), jnp.float32)]
    out_specs = [pl.BlockSpec((nb, c), lambda i: (i, 0))]
    if proj:
        out_shape.append(jax.ShapeDtypeStruct((n, du), jnp.float32))
        out_specs.append(pl.BlockSpec((nb, du), lambda i: (i, 0)))
    return pl.pallas_call(
        body,
        grid=(nblk,),
        in_specs=[
            pl.BlockSpec((nb, din), lambda i: (i, 0)),
            pl.BlockSpec((nb, p), lambda i: (i, 0)),
            pl.BlockSpec((nb, p), lambda i: (nblk + i, 0)),
            pl.BlockSpec((din, c), lambda i: (0, 0)),
            pl.BlockSpec((c, c), lambda i: (0, 0)),
            pl.BlockSpec((1, c), lambda i: (0, 0)),
            pl.BlockSpec((wu.shape[0], du), lambda i: (0, 0)),
            pl.BlockSpec((1, du), lambda i: (0, 0)),
        ],
        out_specs=out_specs,
        out_shape=out_shape,
    )(x, acc, acc, wsc, wcb, bf, wu, bu)


def _fold(p, table, y):
    c = p["Wq"].shape[1]
    t2 = table @ p["Wemb2out"]
    b = jnp.outer(p["bin2k"], p["Wemb2out"].sum(0)) + p["bemb2out"][None, :]
    ek = p["Wemb2out"] @ p["Wkkey"]
    b2 = b @ p["Wkkey"] + p["bkkey"][None, :]
    wa2 = p["Walpha"][c:, 0]
    t2w = t2 @ wa2
    bw = b @ wa2
    klwa = y @ (p["Win2k"] * t2w[:, None]) + bw[None, :]
    mshift = jnp.max(klwa)
    b2p = (b2 + p["bedge"][None, :]) * 0.25
    wbig = jnp.concatenate([ek.T, b2p.T, p["Wedge"].T * 0.25,
                            jnp.zeros((c, ROW - 32), jnp.float32)],
                           axis=1)
    wu = p["Wq"] @ wbig
    bu = (p["bq"] @ wbig).reshape(1, ROW)
    w2ke = jnp.concatenate([p["Win2k"] * 0.25,
                            jnp.ones((8, LK), jnp.float32)], axis=0)
    wo = jnp.concatenate([(p["Wemb2out"] @ wa2)[:, None],
                          p["Wemb2out"]], axis=1)
    bo = jnp.concatenate([(b @ wa2)[:, None] - mshift, b], axis=1)
    wcat = jnp.concatenate([p["Win2k"].T, bo], axis=1)
    wsc = p["Wskip"] @ p["Wcomb"][:c]
    bf = (p["bskip"] @ p["Wcomb"][:c] + p["bcomb"]).reshape(1, c)
    wcb = p["Wcomb"][c:]
    return dict(c=c, tab=table, tabt=table.T, w2ke=w2ke, wo=wo, wcat=wcat,
                wu=wu, bu=bu, wsc=wsc, bf=bf, wcb=wcb)


def kernel(features, edge_index, edge_attr, y, eval_mask, table, layers):
    n = features.shape[0]
    e = edge_index.shape[1]
    nch = e // _CHUNK
    src2d = edge_index[0].reshape(nch, _CHUNK)
    dst2d = edge_index[1].reshape(nch, _CHUNK)
    y128 = jnp.pad(y, ((0, 0), (0, ROW - y.shape[1])))

    folds = [_fold(p, table, y) for p in layers]
    gather = _make_gather(e)
    ys = gather(src2d, y128)

    x = features
    for li, f in enumerate(folds):
        c = f["c"]
        p = ROW
        last = li == len(folds) - 1
        if li == 0:
            u_nodes = _proj_call(features, f["wu"], f["bu"])
        g = gather(dst2d, u_nodes)
        msg = _edge_call(ys, g, edge_attr, f, p)
        acc = _make_scatter(n, e, p)(dst2d, msg, jnp.zeros((n, p), jnp.float32))
        nxt = None if last else folds[li + 1]
        res = _combine_call(x, acc, f["wsc"], f["wcb"], f["bf"],
                            None if last else nxt["wu"],
                            None if last else nxt["bu"],
                            c, p, relu=not last)
        if last:
            x = res[0]
        else:
            x, u_nodes = res
    return x


# speedup vs baseline: 1.2120x; 1.0673x over previous
import functools

import jax
import jax.numpy as jnp
from jax import lax
from jax.experimental import pallas as pl
from jax.experimental.pallas import tpu as pltpu
from jax.experimental.pallas import tpu_sc as plsc

LD = 112
LK = 16
ROW = 128
_CHUNK = 128
_NW = 32


def _sc_mesh():
    return plsc.VectorSubcoreMesh(core_axis_name="c", subcore_axis_name="s")


_K = 6


def _make_gather(e):
    nch = e // _CHUNK
    nj = (nch + _NW - 1) // _NW
    nj_outer = (nj + _K - 1) // _K

    @functools.partial(
        pl.kernel,
        out_type=jax.ShapeDtypeStruct((e, ROW), jnp.float32),
        mesh=_sc_mesh(),
        scratch_types=[
            [pltpu.VMEM((1, _CHUNK), jnp.int32) for _ in range(_K)],
            [pltpu.VMEM((_CHUNK, ROW), jnp.float32) for _ in range(_K)],
            [pltpu.SemaphoreType.DMA for _ in range(_K)],
            [pltpu.SemaphoreType.DMA for _ in range(_K)],
            [pltpu.SemaphoreType.DMA for _ in range(_K)],
        ],
    )
    def gk(idx_hbm, tab_hbm, out_hbm, idx_v, rows_v, si, sg, sw):
        wid = lax.axis_index("s") * 2 + lax.axis_index("c")

        def body(j, carry):
            chs = [wid + _NW * (j * _K + kk) for kk in range(_K)]
            for kk in range(_K):
                @pl.when(chs[kk] < nch)
                def _(kk=kk):
                    pltpu.async_copy(idx_hbm.at[pl.ds(chs[kk], 1)],
                                     idx_v[kk], si[kk])
            for kk in range(_K):
                @pl.when(chs[kk] < nch)
                def _(kk=kk):
                    pltpu.make_async_copy(idx_hbm.at[pl.ds(chs[kk], 1)],
                                          idx_v[kk], si[kk]).wait()
                    pltpu.async_copy(tab_hbm.at[idx_v[kk].at[0]],
                                     rows_v[kk], sg[kk])
            for kk in range(_K):
                @pl.when(chs[kk] < nch)
                def _(kk=kk):
                    pltpu.make_async_copy(tab_hbm.at[idx_v[kk].at[0]],
                                          rows_v[kk], sg[kk]).wait()
                    pltpu.async_copy(
                        rows_v[kk],
                        out_hbm.at[pl.ds(chs[kk] * _CHUNK, _CHUNK)], sw[kk])
            for kk in range(_K):
                @pl.when(chs[kk] < nch)
                def _(kk=kk):
                    pltpu.make_async_copy(
                        rows_v[kk],
                        out_hbm.at[pl.ds(chs[kk] * _CHUNK, _CHUNK)],
                        sw[kk]).wait()
            return carry

        lax.fori_loop(0, nj_outer, body, 0)

    return gk


def _make_scatter(n, e, p=ROW):
    nch = e // _CHUNK
    nj = (nch + _NW - 1) // _NW
    ks = 3
    rpt = (-(-n // 16) + 7) // 8 * 8
    rlast = n - 15 * rpt

    @functools.partial(
        pl.kernel,
        out_type=jax.ShapeDtypeStruct((2 * n, p), jnp.float32),
        mesh=_sc_mesh(),
        scratch_types=[
            [pltpu.VMEM((1, _CHUNK), jnp.int32) for _ in range(ks)],
            [pltpu.VMEM((_CHUNK, p), jnp.float32) for _ in range(ks)],
            pltpu.VMEM_SHARED((n, p), jnp.float32),
            [pltpu.SemaphoreType.DMA for _ in range(ks)],
            [pltpu.SemaphoreType.DMA for _ in range(ks)],
            [pltpu.SemaphoreType.DMA for _ in range(ks)],
        ],
    )
    def sk(idx_hbm, msg_hbm, init_hbm, out_hbm, idx_v, rows_v, acc_sh,
           si, sm, sa):
        cid = lax.axis_index("c")
        sid = lax.axis_index("s")
        wid = sid * 2 + cid

        @pl.when(sid < 15)
        def _():
            pltpu.sync_copy(init_hbm.at[pl.ds(cid * n + sid * rpt, rpt)],
                            acc_sh.at[pl.ds(sid * rpt, rpt)])

        @pl.when(sid == 15)
        def _():
            pltpu.sync_copy(init_hbm.at[pl.ds(cid * n + 15 * rpt, rlast)],
                            acc_sh.at[pl.ds(15 * rpt, rlast)])

        plsc.subcore_barrier()

        def body(j, carry):
            chs = [wid + _NW * (j * ks + kk) for kk in range(ks)]
            for kk in range(ks):
                @pl.when(chs[kk] < nch)
                def _(kk=kk):
                    pltpu.async_copy(idx_hbm.at[pl.ds(chs[kk], 1)],
                                     idx_v[kk], si[kk])
                    pltpu.async_copy(
                        msg_hbm.at[pl.ds(chs[kk] * _CHUNK, _CHUNK)],
                        rows_v[kk], sm[kk])
            for kk in range(ks):
                @pl.when(chs[kk] < nch)
                def _(kk=kk):
                    pltpu.make_async_copy(idx_hbm.at[pl.ds(chs[kk], 1)],
                                          idx_v[kk], si[kk]).wait()
                    pltpu.make_async_copy(
                        msg_hbm.at[pl.ds(chs[kk] * _CHUNK, _CHUNK)],
                        rows_v[kk], sm[kk]).wait()
                    pltpu.async_copy(rows_v[kk], acc_sh.at[idx_v[kk].at[0]],
                                     sa[kk], add=True)
            for kk in range(ks):
                @pl.when(chs[kk] < nch)
                def _(kk=kk):
                    pltpu.make_async_copy(rows_v[kk],
                                          acc_sh.at[idx_v[kk].at[0]],
                                          sa[kk]).wait()
            return carry

        lax.fori_loop(0, (nj + ks - 1) // ks, body, 0)
        plsc.subcore_barrier()

        @pl.when(sid < 15)
        def _():
            pltpu.sync_copy(acc_sh.at[pl.ds(sid * rpt, rpt)],
                            out_hbm.at[pl.ds(cid * n + sid * rpt, rpt)])

        @pl.when(sid == 15)
        def _():
            pltpu.sync_copy(acc_sh.at[pl.ds(15 * rpt, rlast)],
                            out_hbm.at[pl.ds(cid * n + 15 * rpt, rlast)])

    return sk


def _edge_body(ys_ref, g_ref, ea_ref, tabt_ref, tab_ref, w2ke_ref,
               wcat_ref, wo_ref, msg_ref, *, c, p):
    ysv = ys_ref[:, 0:LD]
    g = g_ref[...]
    z = g[:, 0:8]
    qb = g[:, 8:8 + LK]
    ve = g[:, 8 + LK:8 + LK + 8]
    u = jnp.dot(z, tabt_ref[...], preferred_element_type=jnp.float32)
    x2 = jnp.concatenate([ysv * u, ea_ref[...] * ve], axis=1)
    xl = jnp.dot(x2, w2ke_ref[...], preferred_element_type=jnp.float32) + qb
    exl = jnp.exp(xl)
    alpha = exl / jnp.sum(exl, axis=1, keepdims=True)
    rab = jnp.dot(alpha, wcat_ref[...], preferred_element_type=jnp.float32)
    r = rab[:, 0:LD]
    h8 = jnp.dot(ysv * r, tab_ref[...], preferred_element_type=jnp.float32)
    os = (jnp.dot(h8, wo_ref[...], preferred_element_type=jnp.float32)
          + rab[:, LD:LD + 1 + c])
    w = jnp.exp(os[:, 0:1])
    lane = jax.lax.broadcasted_iota(jnp.int32, (os.shape[0], 1 + c), 1)
    base = jnp.where(lane == 0, 1.0, os)
    msg_ref[:, 0:1 + c] = w * base
    msg_ref[:, 1 + c:p] = jnp.zeros((os.shape[0], p - 1 - c), jnp.float32)


def _edge_call(ys, g, ea, f, p, off, eb=8000):
    e = g.shape[0]
    c = f["c"]
    return pl.pallas_call(
        functools.partial(_edge_body, c=c, p=p),
        grid=(e // eb,),
        in_specs=[
            pl.BlockSpec((eb, ROW), lambda i, o=off: (o + i, 0)),
            pl.BlockSpec((eb, ROW), lambda i: (i, 0)),
            pl.BlockSpec((eb, 8), lambda i, o=off: (o + i, 0)),
            pl.BlockSpec((8, LD), lambda i: (0, 0)),
            pl.BlockSpec((LD, 8), lambda i: (0, 0)),
            pl.BlockSpec((LD + 8, LK), lambda i: (0, 0)),
            pl.BlockSpec((LK, LD + 1 + c), lambda i: (0, 0)),
            pl.BlockSpec((8, 1 + c), lambda i: (0, 0)),
        ],
        out_specs=pl.BlockSpec((eb, p), lambda i: (i, 0)),
        out_shape=jax.ShapeDtypeStruct((e, p), jnp.float32),
    )(ys, g, ea, f["tabt"], f["tab"], f["w2ke"], f["wcat"], f["wo"])


def _proj_body(x_ref, w_ref, b_ref, o_ref):
    o_ref[...] = (jnp.dot(x_ref[...], w_ref[...],
                          preferred_element_type=jnp.float32) + b_ref[...])


def _proj_call(x, w, b, nb=2000):
    n, din = x.shape
    d = w.shape[1]
    return pl.pallas_call(
        _proj_body,
        grid=(n // nb,),
        in_specs=[
            pl.BlockSpec((nb, din), lambda i: (i, 0)),
            pl.BlockSpec((din, d), lambda i: (0, 0)),
            pl.BlockSpec((1, d), lambda i: (0, 0)),
        ],
        out_specs=pl.BlockSpec((nb, d), lambda i: (i, 0)),
        out_shape=jax.ShapeDtypeStruct((n, d), jnp.float32),
    )(x, w, b)


def _combine_call(x, acc, wsc, wcb, bf, wu, bu, c, p, relu, nb=2000):
    n, din = x.shape
    proj = wu is not None
    if not proj:
        wu = jnp.zeros((c, 8), jnp.float32)
        bu = jnp.zeros((1, 8), jnp.float32)
    du = wu.shape[1]
    nblk = n // nb

    def body(x_ref, a0_ref, a1_ref, wsc_ref, wcb_ref, bf_ref, wu_ref, bu_ref,
             *outs):
        den = a0_ref[:, 0:1] + a1_ref[:, 0:1]
        num = a0_ref[:, 1:c + 1] + a1_ref[:, 1:c + 1]
        m = num / (den + 1e-16)
        xn = (jnp.dot(x_ref[...], wsc_ref[...],
                      preferred_element_type=jnp.float32)
              + jnp.dot(m, wcb_ref[...], preferred_element_type=jnp.float32)
              + bf_ref[...])
        if relu:
            xn = jnp.maximum(xn, 0.0)
        outs[0][...] = xn
        if proj:
            outs[1][...] = (jnp.dot(xn, wu_ref[...],
                                    preferred_element_type=jnp.float32)
                            + bu_ref[...])

    out_shape = [jax.ShapeDtypeStruct((n, c), jnp.float32)]
    out_specs = [pl.BlockSpec((nb, c), lambda i: (i, 0))]
    if proj:
        out_shape.append(jax.ShapeDtypeStruct((n, du), jnp.float32))
        out_specs.append(pl.BlockSpec((nb, du), lambda i: (i, 0)))
    return pl.pallas_call(
        body,
        grid=(nblk,),
        in_specs=[
            pl.BlockSpec((nb, din), lambda i: (i, 0)),
            pl.BlockSpec((nb, p), lambda i: (i, 0)),
            pl.BlockSpec((nb, p), lambda i: (nblk + i, 0)),
            pl.BlockSpec((din, c), lambda i: (0, 0)),
            pl.BlockSpec((c, c), lambda i: (0, 0)),
            pl.BlockSpec((1, c), lambda i: (0, 0)),
            pl.BlockSpec((wu.shape[0], du), lambda i: (0, 0)),
            pl.BlockSpec((1, du), lambda i: (0, 0)),
        ],
        out_specs=out_specs,
        out_shape=out_shape,
    )(x, acc, acc, wsc, wcb, bf, wu, bu)


def _fold(p, table, y):
    c = p["Wq"].shape[1]
    t2 = table @ p["Wemb2out"]
    b = jnp.outer(p["bin2k"], p["Wemb2out"].sum(0)) + p["bemb2out"][None, :]
    ek = p["Wemb2out"] @ p["Wkkey"]
    b2 = b @ p["Wkkey"] + p["bkkey"][None, :]
    wa2 = p["Walpha"][c:, 0]
    t2w = t2 @ wa2
    bw = b @ wa2
    klwa = y @ (p["Win2k"] * t2w[:, None]) + bw[None, :]
    mshift = jnp.max(klwa)
    b2p = (b2 + p["bedge"][None, :]) * 0.25
    wbig = jnp.concatenate([ek.T, b2p.T, p["Wedge"].T * 0.25,
                            jnp.zeros((c, ROW - 32), jnp.float32)],
                           axis=1)
    wu = p["Wq"] @ wbig
    bu = (p["bq"] @ wbig).reshape(1, ROW)
    w2ke = jnp.concatenate([p["Win2k"] * 0.25,
                            jnp.ones((8, LK), jnp.float32)], axis=0)
    wo = jnp.concatenate([(p["Wemb2out"] @ wa2)[:, None],
                          p["Wemb2out"]], axis=1)
    bo = jnp.concatenate([(b @ wa2)[:, None] - mshift, b], axis=1)
    wcat = jnp.concatenate([p["Win2k"].T, bo], axis=1)
    wsc = p["Wskip"] @ p["Wcomb"][:c]
    bf = (p["bskip"] @ p["Wcomb"][:c] + p["bcomb"]).reshape(1, c)
    wcb = p["Wcomb"][c:]
    return dict(c=c, tab=table, tabt=table.T, w2ke=w2ke, wo=wo, wcat=wcat,
                wu=wu, bu=bu, wsc=wsc, bf=bf, wcb=wcb)


def kernel(features, edge_index, edge_attr, y, eval_mask, table, layers):
    n = features.shape[0]
    e = edge_index.shape[1]
    nch = e // _CHUNK
    src2d = edge_index[0].reshape(nch, _CHUNK)
    dst2d = edge_index[1].reshape(nch, _CHUNK)
    y128 = jnp.pad(y, ((0, 0), (0, ROW - y.shape[1])))

    folds = [_fold(p, table, y) for p in layers]
    gather = _make_gather(e)
    ys = gather(src2d, y128)

    eh = e // 2
    nchh = nch // 2
    gather_h = _make_gather(eh)
    scatter_h = _make_scatter(n, eh)
    dst_h = [dst2d[:nchh], dst2d[nchh:]]

    x = features
    for li, f in enumerate(folds):
        c = f["c"]
        p = ROW
        last = li == len(folds) - 1
        if li == 0:
            u_nodes = _proj_call(features, f["wu"], f["bu"])
        acc = jnp.zeros((2 * n, p), jnp.float32)
        g = [gather_h(dst_h[h], u_nodes) for h in range(2)]
        for h in range(2):
            msg = _edge_call(ys, g[h], edge_attr, f, p,
                             h * (eh // 8000))
            acc = scatter_h(dst_h[h], msg, acc)
        nxt = None if last else folds[li + 1]
        res = _combine_call(x, acc, f["wsc"], f["wcb"], f["bf"],
                            None if last else nxt["wu"],
                            None if last else nxt["bu"],
                            c, p, relu=not last)
        if last:
            x = res[0]
        else:
            x, u_nodes = res
    return x


# speedup vs baseline: 1.2134x; 1.0012x over previous
import functools

import jax
import jax.numpy as jnp
from jax import lax
from jax.experimental import pallas as pl
from jax.experimental.pallas import tpu as pltpu
from jax.experimental.pallas import tpu_sc as plsc

LD = 112
LK = 16
ROW = 128
_CHUNK = 128
_NW = 32


def _sc_mesh():
    return plsc.VectorSubcoreMesh(core_axis_name="c", subcore_axis_name="s")


_K = 6


def _make_gather(e):
    nch = e // _CHUNK
    nj = (nch + _NW - 1) // _NW
    nj_outer = (nj + _K - 1) // _K

    @functools.partial(
        pl.kernel,
        out_type=jax.ShapeDtypeStruct((e, ROW), jnp.float32),
        mesh=_sc_mesh(),
        scratch_types=[
            [pltpu.VMEM((1, _CHUNK), jnp.int32) for _ in range(_K)],
            [pltpu.VMEM((_CHUNK, ROW), jnp.float32) for _ in range(_K)],
            [pltpu.SemaphoreType.DMA for _ in range(_K)],
            [pltpu.SemaphoreType.DMA for _ in range(_K)],
            [pltpu.SemaphoreType.DMA for _ in range(_K)],
        ],
    )
    def gk(idx_hbm, tab_hbm, out_hbm, idx_v, rows_v, si, sg, sw):
        wid = lax.axis_index("s") * 2 + lax.axis_index("c")

        def body(j, carry):
            chs = [wid + _NW * (j * _K + kk) for kk in range(_K)]
            for kk in range(_K):
                @pl.when(chs[kk] < nch)
                def _(kk=kk):
                    pltpu.async_copy(idx_hbm.at[pl.ds(chs[kk], 1)],
                                     idx_v[kk], si[kk])
            for kk in range(_K):
                @pl.when(chs[kk] < nch)
                def _(kk=kk):
                    pltpu.make_async_copy(idx_hbm.at[pl.ds(chs[kk], 1)],
                                          idx_v[kk], si[kk]).wait()
                    pltpu.async_copy(tab_hbm.at[idx_v[kk].at[0]],
                                     rows_v[kk], sg[kk])
            for kk in range(_K):
                @pl.when(chs[kk] < nch)
                def _(kk=kk):
                    pltpu.make_async_copy(tab_hbm.at[idx_v[kk].at[0]],
                                          rows_v[kk], sg[kk]).wait()
                    pltpu.async_copy(
                        rows_v[kk],
                        out_hbm.at[pl.ds(chs[kk] * _CHUNK, _CHUNK)], sw[kk])
            for kk in range(_K):
                @pl.when(chs[kk] < nch)
                def _(kk=kk):
                    pltpu.make_async_copy(
                        rows_v[kk],
                        out_hbm.at[pl.ds(chs[kk] * _CHUNK, _CHUNK)],
                        sw[kk]).wait()
            return carry

        lax.fori_loop(0, nj_outer, body, 0)

    return gk


def _make_scatter(n, e, p=ROW):
    nch = e // _CHUNK
    nj = (nch + _NW - 1) // _NW
    ks = 3
    rpt = (-(-n // 16) + 7) // 8 * 8
    rlast = n - 15 * rpt

    @functools.partial(
        pl.kernel,
        out_type=jax.ShapeDtypeStruct((2 * n, p), jnp.float32),
        mesh=_sc_mesh(),
        scratch_types=[
            [pltpu.VMEM((1, _CHUNK), jnp.int32) for _ in range(ks)],
            [pltpu.VMEM((_CHUNK, p), jnp.float32) for _ in range(ks)],
            pltpu.VMEM_SHARED((n, p), jnp.float32),
            [pltpu.SemaphoreType.DMA for _ in range(ks)],
            [pltpu.SemaphoreType.DMA for _ in range(ks)],
            [pltpu.SemaphoreType.DMA for _ in range(ks)],
        ],
    )
    def sk(idx_hbm, msg_hbm, init_hbm, out_hbm, idx_v, rows_v, acc_sh,
           si, sm, sa):
        cid = lax.axis_index("c")
        sid = lax.axis_index("s")
        wid = sid * 2 + cid

        @pl.when(sid < 15)
        def _():
            pltpu.sync_copy(init_hbm.at[pl.ds(cid * n + sid * rpt, rpt)],
                            acc_sh.at[pl.ds(sid * rpt, rpt)])

        @pl.when(sid == 15)
        def _():
            pltpu.sync_copy(init_hbm.at[pl.ds(cid * n + 15 * rpt, rlast)],
                            acc_sh.at[pl.ds(15 * rpt, rlast)])

        plsc.subcore_barrier()

        def body(j, carry):
            chs = [wid + _NW * (j * ks + kk) for kk in range(ks)]
            for kk in range(ks):
                @pl.when(chs[kk] < nch)
                def _(kk=kk):
                    pltpu.async_copy(idx_hbm.at[pl.ds(chs[kk], 1)],
                                     idx_v[kk], si[kk])
                    pltpu.async_copy(
                        msg_hbm.at[pl.ds(chs[kk] * _CHUNK, _CHUNK)],
                        rows_v[kk], sm[kk])
            for kk in range(ks):
                @pl.when(chs[kk] < nch)
                def _(kk=kk):
                    pltpu.make_async_copy(idx_hbm.at[pl.ds(chs[kk], 1)],
                                          idx_v[kk], si[kk]).wait()
                    pltpu.make_async_copy(
                        msg_hbm.at[pl.ds(chs[kk] * _CHUNK, _CHUNK)],
                        rows_v[kk], sm[kk]).wait()
                    pltpu.async_copy(rows_v[kk], acc_sh.at[idx_v[kk].at[0]],
                                     sa[kk], add=True)
            for kk in range(ks):
                @pl.when(chs[kk] < nch)
                def _(kk=kk):
                    pltpu.make_async_copy(rows_v[kk],
                                          acc_sh.at[idx_v[kk].at[0]],
                                          sa[kk]).wait()
            return carry

        lax.fori_loop(0, (nj + ks - 1) // ks, body, 0)
        plsc.subcore_barrier()

        @pl.when(sid < 15)
        def _():
            pltpu.sync_copy(acc_sh.at[pl.ds(sid * rpt, rpt)],
                            out_hbm.at[pl.ds(cid * n + sid * rpt, rpt)])

        @pl.when(sid == 15)
        def _():
            pltpu.sync_copy(acc_sh.at[pl.ds(15 * rpt, rlast)],
                            out_hbm.at[pl.ds(cid * n + 15 * rpt, rlast)])

    return sk


def _edge_body(ys_ref, g_ref, ea_ref, tabt_ref, tab_ref, w2ke_ref,
               wcat_ref, wo_ref, msg_ref, *, c, p):
    ysv = ys_ref[:, 0:LD]
    g = g_ref[...]
    z = g[:, 0:8]
    qb = g[:, 8:8 + LK]
    ve = g[:, 8 + LK:8 + LK + 8]
    u = jnp.dot(z, tabt_ref[...], preferred_element_type=jnp.float32)
    x2 = jnp.concatenate([ysv * u, ea_ref[...] * ve], axis=1)
    xl = jnp.dot(x2, w2ke_ref[...], preferred_element_type=jnp.float32) + qb
    exl = jnp.exp(xl)
    alpha = exl / jnp.sum(exl, axis=1, keepdims=True)
    rab = jnp.dot(alpha, wcat_ref[...], preferred_element_type=jnp.float32)
    r = rab[:, 0:LD]
    h8 = jnp.dot(ysv * r, tab_ref[...], preferred_element_type=jnp.float32)
    os = (jnp.dot(h8, wo_ref[...], preferred_element_type=jnp.float32)
          + rab[:, LD:LD + 1 + c])
    w = jnp.exp(os[:, 0:1])
    lane = jax.lax.broadcasted_iota(jnp.int32, (os.shape[0], 1 + c), 1)
    base = jnp.where(lane == 0, 1.0, os)
    msg_ref[:, 0:1 + c] = w * base
    msg_ref[:, 1 + c:p] = jnp.zeros((os.shape[0], p - 1 - c), jnp.float32)


def _edge_call(ys, g, ea, f, p, off, eb=8000):
    e = g.shape[0]
    c = f["c"]
    return pl.pallas_call(
        functools.partial(_edge_body, c=c, p=p),
        grid=(e // eb,),
        in_specs=[
            pl.BlockSpec((eb, ROW), lambda i, o=off: (o + i, 0)),
            pl.BlockSpec((eb, ROW), lambda i: (i, 0)),
            pl.BlockSpec((eb, 8), lambda i, o=off: (o + i, 0)),
            pl.BlockSpec((8, LD), lambda i: (0, 0)),
            pl.BlockSpec((LD, 8), lambda i: (0, 0)),
            pl.BlockSpec((LD + 8, LK), lambda i: (0, 0)),
            pl.BlockSpec((LK, LD + 1 + c), lambda i: (0, 0)),
            pl.BlockSpec((8, 1 + c), lambda i: (0, 0)),
        ],
        out_specs=pl.BlockSpec((eb, p), lambda i: (i, 0)),
        out_shape=jax.ShapeDtypeStruct((e, p), jnp.float32),
    )(ys, g, ea, f["tabt"], f["tab"], f["w2ke"], f["wcat"], f["wo"])


def _pad_body(x_ref, o_ref, *, d):
    o_ref[...] = jnp.concatenate(
        [x_ref[...],
         jnp.zeros((x_ref.shape[0], ROW - d), jnp.float32)], axis=1)


def _pad_call(x, nb=2000):
    n, d = x.shape
    return pl.pallas_call(
        functools.partial(_pad_body, d=d),
        grid=(n // nb,),
        in_specs=[pl.BlockSpec((nb, d), lambda i: (i, 0))],
        out_specs=pl.BlockSpec((nb, ROW), lambda i: (i, 0)),
        out_shape=jax.ShapeDtypeStruct((n, ROW), jnp.float32),
    )(x)


def _proj_body(x_ref, w_ref, b_ref, o_ref):
    o_ref[...] = (jnp.dot(x_ref[...], w_ref[...],
                          preferred_element_type=jnp.float32) + b_ref[...])


def _proj_call(x, w, b, nb=2000):
    n, din = x.shape
    d = w.shape[1]
    return pl.pallas_call(
        _proj_body,
        grid=(n // nb,),
        in_specs=[
            pl.BlockSpec((nb, din), lambda i: (i, 0)),
            pl.BlockSpec((din, d), lambda i: (0, 0)),
            pl.BlockSpec((1, d), lambda i: (0, 0)),
        ],
        out_specs=pl.BlockSpec((nb, d), lambda i: (i, 0)),
        out_shape=jax.ShapeDtypeStruct((n, d), jnp.float32),
    )(x, w, b)


def _combine_call(x, acc, wsc, wcb, bf, wu, bu, c, p, relu, nb=2000):
    n, din = x.shape
    proj = wu is not None
    if not proj:
        wu = jnp.zeros((c, 8), jnp.float32)
        bu = jnp.zeros((1, 8), jnp.float32)
    du = wu.shape[1]
    nblk = n // nb

    def body(x_ref, a0_ref, a1_ref, wsc_ref, wcb_ref, bf_ref, wu_ref, bu_ref,
             *outs):
        den = a0_ref[:, 0:1] + a1_ref[:, 0:1]
        num = a0_ref[:, 1:c + 1] + a1_ref[:, 1:c + 1]
        m = num / (den + 1e-16)
        xn = (jnp.dot(x_ref[...], wsc_ref[...],
                      preferred_element_type=jnp.float32)
              + jnp.dot(m, wcb_ref[...], preferred_element_type=jnp.float32)
              + bf_ref[...])
        if relu:
            xn = jnp.maximum(xn, 0.0)
        outs[0][...] = xn
        if proj:
            outs[1][...] = (jnp.dot(xn, wu_ref[...],
                                    preferred_element_type=jnp.float32)
                            + bu_ref[...])

    out_shape = [jax.ShapeDtypeStruct((n, c), jnp.float32)]
    out_specs = [pl.BlockSpec((nb, c), lambda i: (i, 0))]
    if proj:
        out_shape.append(jax.ShapeDtypeStruct((n, du), jnp.float32))
        out_specs.append(pl.BlockSpec((nb, du), lambda i: (i, 0)))
    return pl.pallas_call(
        body,
        grid=(nblk,),
        in_specs=[
            pl.BlockSpec((nb, din), lambda i: (i, 0)),
            pl.BlockSpec((nb, p), lambda i: (i, 0)),
            pl.BlockSpec((nb, p), lambda i: (nblk + i, 0)),
            pl.BlockSpec((din, c), lambda i: (0, 0)),
            pl.BlockSpec((c, c), lambda i: (0, 0)),
            pl.BlockSpec((1, c), lambda i: (0, 0)),
            pl.BlockSpec((wu.shape[0], du), lambda i: (0, 0)),
            pl.BlockSpec((1, du), lambda i: (0, 0)),
        ],
        out_specs=out_specs,
        out_shape=out_shape,
    )(x, acc, acc, wsc, wcb, bf, wu, bu)


def _fold(p, table, y):
    c = p["Wq"].shape[1]
    t2 = table @ p["Wemb2out"]
    b = jnp.outer(p["bin2k"], p["Wemb2out"].sum(0)) + p["bemb2out"][None, :]
    ek = p["Wemb2out"] @ p["Wkkey"]
    b2 = b @ p["Wkkey"] + p["bkkey"][None, :]
    wa2 = p["Walpha"][c:, 0]
    t2w = t2 @ wa2
    bw = b @ wa2
    klwa = y @ (p["Win2k"] * t2w[:, None]) + bw[None, :]
    mshift = jnp.max(klwa)
    b2p = (b2 + p["bedge"][None, :]) * 0.25
    wbig = jnp.concatenate([ek.T, b2p.T, p["Wedge"].T * 0.25,
                            jnp.zeros((c, ROW - 32), jnp.float32)],
                           axis=1)
    wu = p["Wq"] @ wbig
    bu = (p["bq"] @ wbig).reshape(1, ROW)
    w2ke = jnp.concatenate([p["Win2k"] * 0.25,
                            jnp.ones((8, LK), jnp.float32)], axis=0)
    wo = jnp.concatenate([(p["Wemb2out"] @ wa2)[:, None],
                          p["Wemb2out"]], axis=1)
    bo = jnp.concatenate([(b @ wa2)[:, None] - mshift, b], axis=1)
    wcat = jnp.concatenate([p["Win2k"].T, bo], axis=1)
    wsc = p["Wskip"] @ p["Wcomb"][:c]
    bf = (p["bskip"] @ p["Wcomb"][:c] + p["bcomb"]).reshape(1, c)
    wcb = p["Wcomb"][c:]
    return dict(c=c, tab=table, tabt=table.T, w2ke=w2ke, wo=wo, wcat=wcat,
                wu=wu, bu=bu, wsc=wsc, bf=bf, wcb=wcb)


def kernel(features, edge_index, edge_attr, y, eval_mask, table, layers):
    n = features.shape[0]
    e = edge_index.shape[1]
    nch = e // _CHUNK
    src2d = edge_index[0].reshape(nch, _CHUNK)
    dst2d = edge_index[1].reshape(nch, _CHUNK)
    y128 = _pad_call(y)

    folds = [_fold(p, table, y) for p in layers]
    gather = _make_gather(e)
    ys = gather(src2d, y128)

    eh = e // 2
    nchh = nch // 2
    gather_h = _make_gather(eh)
    scatter_h = _make_scatter(n, eh)
    dst_h = [dst2d[:nchh], dst2d[nchh:]]

    x = features
    for li, f in enumerate(folds):
        c = f["c"]
        p = ROW
        last = li == len(folds) - 1
        if li == 0:
            u_nodes = _proj_call(features, f["wu"], f["bu"])
        acc = jnp.zeros((2 * n, p), jnp.float32)
        g = [gather_h(dst_h[h], u_nodes) for h in range(2)]
        for h in range(2):
            msg = _edge_call(ys, g[h], edge_attr, f, p,
                             h * (eh // 8000))
            acc = scatter_h(dst_h[h], msg, acc)
        nxt = None if last else folds[li + 1]
        res = _combine_call(x, acc, f["wsc"], f["wcb"], f["bf"],
                            None if last else nxt["wu"],
                            None if last else nxt["bu"],
                            c, p, relu=not last)
        if last:
            x = res[0]
        else:
            x, u_nodes = res
    return x
